# baseline (speedup 1.0000x reference)
import jax
import jax.numpy as jnp
import numpy as np
from jax import lax
from jax.experimental import pallas as pl
from jax.experimental.pallas import tpu as pltpu

_W = 200
_H = 200
_A = 9
_NH = 5
_G = 20
_N = _NH * _A * _W * _H
_PER_HEAD = _A * _W * _H
_BIG = 1 << 30
_C7 = float(np.float32(0.7 / 1.7))
_C3 = float(np.float32(0.3 / 1.3))


_WC = 8


def _main_body(gt_ref, cls_ref, reg_ref, ints_ref, floats_ref):
    i = pl.program_id(0)
    head = i // _A
    a = i - head * _A

    gts = [[gt_ref[g, c] for c in range(4)] for g in range(_G)]
    area_g = [(gts[g][2] - gts[g][0]) * (gts[g][3] - gts[g][1]) for g in range(_G)]

    w2d = lax.broadcasted_iota(jnp.int32, (_WC, _H), 0)
    h2d = lax.broadcasted_iota(jnp.int32, (_WC, _H), 1)
    base_off = head * _PER_HEAD + a

    winners = [None] * _G
    sum_b = None
    npos_s = None
    nzero_s = None
    fb_s = None
    for ws in range(0, _W, _WC):
        sl = slice(ws, ws + _WC)
        p = cls_ref[0, 0, 0, sl, :]
        bx1 = reg_ref[0, 0, 0, sl, :]
        by1 = reg_ref[0, 0, 1, sl, :]
        bx2 = reg_ref[0, 0, 2, sl, :]
        by2 = reg_ref[0, 0, 3, sl, :]
        area_b = (bx2 - bx1) * (by2 - by1)
        n_c = ((w2d + ws) * _H + h2d) * _A + base_off

        jf = None
        nviol = None
        for g in range(_G):
            gx1, gy1, gx2, gy2 = gts[g]
            iw = jnp.maximum(jnp.minimum(bx2, gx2) - jnp.maximum(bx1, gx1), 0.0)
            ih = jnp.maximum(jnp.minimum(by2, gy2) - jnp.maximum(by1, gy1), 0.0)
            inter = iw * ih
            ab = area_b + area_g[g]
            upos = inter < ab
            pos = (inter >= _C7 * ab) & upos
            viol = (inter >= _C3 * ab) & upos
            inc = jnp.where(viol, 1, 0)
            nviol = inc if nviol is None else (nviol + inc)
            reach = (jf >= g) if jf is not None else None
            cand = (inter > 0.0) & upos
            if reach is not None:
                cand = cand & reach
            m_cg = jnp.max(jnp.where(cand, n_c, -1))
            winners[g] = m_cg if winners[g] is None else \
                jnp.maximum(winners[g], m_cg)
            posg = jnp.where(pos, g, _G)
            jf = posg if jf is None else jnp.minimum(jf, posg)

        anypos = jf < _G
        zerolab = (jf >= _G) & (nviol == 0)
        basekeep = anypos | zerolab
        bce = jnp.where(anypos, -jnp.log(p), 0.0) + jnp.where(
            zerolab, -jnp.log(1.0 - p), 0.0)
        t = jnp.sum(bce)
        sum_b = t if sum_b is None else (sum_b + t)
        t = jnp.sum(jnp.where(anypos, 1, 0))
        npos_s = t if npos_s is None else (npos_s + t)
        t = jnp.sum(jnp.where(zerolab, 1, 0))
        nzero_s = t if nzero_s is None else (nzero_s + t)
        t = jnp.min(jnp.where(basekeep, n_c, _BIG))
        fb_s = t if fb_s is None else jnp.minimum(fb_s, t)

    for g in range(_G):
        ints_ref[0, 0, g] = winners[g]
    ints_ref[0, 0, _G] = fb_s
    ints_ref[0, 0, _G + 1] = npos_s
    ints_ref[0, 0, _G + 2] = nzero_s
    floats_ref[0, 0, 0] = sum_b


def _combine_body(ints_ref, floats_ref, pref_ref, fvals_ref):
    nblk = _NH * _A
    iv = ints_ref[...].reshape(nblk, 32)
    fv = floats_ref[...].reshape(nblk, 8)
    maxv = jnp.max(iv, axis=0, keepdims=True)
    minv = jnp.min(iv, axis=0, keepdims=True)
    sumv = jnp.sum(iv, axis=0, keepdims=True)
    sumf = jnp.sum(fv, axis=0, keepdims=True)
    for g in range(_G):
        pref_ref[g] = jnp.maximum(maxv[0, g], 0)
    fb = minv[0, _G]
    pref_ref[_G] = jnp.minimum(fb, _N - 1)
    pref_ref[_G + 1] = fb
    pref_ref[_G + 2] = sumv[0, _G + 1] + sumv[0, _G + 2]
    fvals_ref[0, 0] = sumf[0, 0]


def _epi_body(pref_ref, gt_ref, fvals_ref, cls_ref, reg_ref, out_ref, scr_f):
    k_step = pl.program_id(0)

    gts = [[gt_ref[g, c] for c in range(4)] for g in range(_G)]
    area_g = [(gts[g][2] - gts[g][0]) * (gts[g][3] - gts[g][1]) for g in range(_G)]

    w2d = lax.broadcasted_iota(jnp.int32, (_W, _H), 0)
    h2d = lax.broadcasted_iota(jnp.int32, (_W, _H), 1)
    whbase = w2d * _H + h2d
    idx = pref_ref[k_step]
    rem = idx - (idx // _PER_HEAD) * _PER_HEAD
    wh = rem // _A
    eq = whbase == wh
    for c in range(4):
        scr_f[8 + 4 * k_step + c] = jnp.sum(jnp.where(eq, reg_ref[0, 0, c], 0.0))

    @pl.when(k_step < _G)
    def _extract_p():
        p = jnp.sum(jnp.where(eq, cls_ref[0, 0, 0], 0.0))
        pv = jnp.full((1, 128), p, dtype=jnp.float32)
        scr_f[100 + k_step] = -jnp.log(pv)[0, 0]
        scr_f[128 + k_step] = -jnp.log(1.0 - pv)[0, 0]

    @pl.when(k_step == _G)
    def _assemble():
        w_use = [pref_ref[k] for k in range(_G)]
        fb = pref_ref[_G + 1]
        sum_b = fvals_ref[0, 0]
        nk_extra = None
        for g in range(_G):
            dup = None
            for gp in range(g):
                t = w_use[g] == w_use[gp]
                dup = t if dup is None else (dup | t)
            bx1 = scr_f[8 + 4 * g + 0]
            by1 = scr_f[8 + 4 * g + 1]
            bx2 = scr_f[8 + 4 * g + 2]
            by2 = scr_f[8 + 4 * g + 3]
            area_b = (bx2 - bx1) * (by2 - by1)
            anypos = None
            alllow = None
            for j in range(_G):
                gx1, gy1, gx2, gy2 = gts[j]
                iw = jnp.maximum(jnp.minimum(bx2, gx2) - jnp.maximum(bx1, gx1), 0.0)
                ih = jnp.maximum(jnp.minimum(by2, gy2) - jnp.maximum(by1, gy1), 0.0)
                inter = iw * ih
                ab = area_b + area_g[j]
                nupos = inter >= ab
                pos = (inter >= _C7 * ab) & (~nupos)
                low = (inter < _C3 * ab) | nupos
                anypos = pos if anypos is None else (anypos | pos)
                alllow = low if alllow is None else (alllow & low)
            zerolab = (~anypos) & alllow
            add1 = (~anypos) & (~zerolab)
            repl = zerolab
            if dup is not None:
                add1 = (~dup) & add1
                repl = (~dup) & repl
            lp = scr_f[100 + g]
            l1m = scr_f[128 + g]
            sum_b = sum_b + jnp.where(add1, lp, 0.0) + jnp.where(
                repl, lp - l1m, 0.0)
            inc = jnp.where(add1, 1, 0)
            nk_extra = inc if nk_extra is None else (nk_extra + inc)

        n_keep = (pref_ref[_G + 2] + nk_extra).astype(jnp.float32)
        num = jnp.full((1, 128), sum_b, dtype=jnp.float32)
        den = jnp.full((1, 128), n_keep, dtype=jnp.float32)
        cls_loss = (num / den)[0, 0]

        first_idx = fb
        for g in range(_G):
            first_idx = jnp.minimum(first_idx, w_use[g])
        box0 = [None] * 4
        found = None
        cands = [(w_use[g], g) for g in range(_G)] + [(fb, _G)]
        for idx_k, k in cands:
            sel = idx_k == first_idx
            if found is not None:
                sel = sel & (~found)
            for c in range(4):
                v = scr_f[8 + 4 * k + c]
                box0[c] = jnp.where(sel, v, 0.0) if box0[c] is None else \
                    jnp.where(sel, v, box0[c])
            found = sel if found is None else (found | sel)

        acc = None
        for j in range(_G):
            for c in range(4):
                d = jnp.abs(box0[c] - gts[j][c])
                t = jnp.where(d < 1.0, 0.5 * d * d, d - 0.5)
                acc = t if acc is None else (acc + t)
        reg_loss = acc * (1.0 / (4 * _G))
        out_ref[0, 0] = cls_loss * (1.0 / 256.0) + reg_loss * (1.0 / 2400.0)


def kernel(cls_heads, reg_heads, gt_boxes, interpret=False):
    nblk = _NH * _A
    ints, floats = pl.pallas_call(
        _main_body,
        grid=(nblk,),
        in_specs=[
            pl.BlockSpec(memory_space=pltpu.SMEM),
            pl.BlockSpec((1, 1, 1, _W, _H), lambda i: (i // _A, 0, i % _A, 0, 0)),
            pl.BlockSpec((1, 1, 4, _W, _H), lambda i: (i // _A, 0, i % _A, 0, 0)),
        ],
        out_specs=[
            pl.BlockSpec((1, 1, 32), lambda i: (i, 0, 0), memory_space=pltpu.SMEM),
            pl.BlockSpec((1, 1, 8), lambda i: (i, 0, 0), memory_space=pltpu.SMEM),
        ],
        out_shape=[
            jax.ShapeDtypeStruct((nblk, 1, 32), jnp.int32),
            jax.ShapeDtypeStruct((nblk, 1, 8), jnp.float32),
        ],
        compiler_params=pltpu.CompilerParams(
            dimension_semantics=("arbitrary",),
        ),
        name="rpn_main",
        interpret=interpret,
    )(gt_boxes, cls_heads, reg_heads)

    pref, fvals = pl.pallas_call(
        _combine_body,
        in_specs=[
            pl.BlockSpec(memory_space=pltpu.VMEM),
            pl.BlockSpec(memory_space=pltpu.VMEM),
        ],
        out_specs=[
            pl.BlockSpec(memory_space=pltpu.SMEM),
            pl.BlockSpec(memory_space=pltpu.SMEM),
        ],
        out_shape=[
            jax.ShapeDtypeStruct((32,), jnp.int32),
            jax.ShapeDtypeStruct((1, 8), jnp.float32),
        ],
        name="rpn_combine",
        interpret=interpret,
    )(ints, floats)

    def _cls_map(k, pref):
        idx = pref[k]
        return (idx // _PER_HEAD, 0, (idx - (idx // _PER_HEAD) * _PER_HEAD) % _A,
                0, 0)

    out = pl.pallas_call(
        _epi_body,
        grid_spec=pltpu.PrefetchScalarGridSpec(
            num_scalar_prefetch=1,
            grid=(_G + 1,),
            in_specs=[
                pl.BlockSpec(memory_space=pltpu.SMEM),
                pl.BlockSpec(memory_space=pltpu.SMEM),
                pl.BlockSpec((1, 1, 1, _W, _H), _cls_map),
                pl.BlockSpec((1, 1, 4, _W, _H), _cls_map),
            ],
            out_specs=pl.BlockSpec((1, 1), lambda k, pref: (0, 0),
                                   memory_space=pltpu.SMEM),
            scratch_shapes=[pltpu.SMEM((256,), jnp.float32)],
        ),
        out_shape=jax.ShapeDtypeStruct((1, 1), jnp.float32),
        compiler_params=pltpu.CompilerParams(
            dimension_semantics=("arbitrary",),
        ),
        name="rpn_epilogue",
        interpret=interpret,
    )(pref, gt_boxes, fvals, cls_heads, reg_heads)
    return out[0, 0]


# speedup vs baseline: 1.1019x; 1.1019x over previous
import jax
import jax.numpy as jnp
import numpy as np
from jax import lax
from jax.experimental import pallas as pl
from jax.experimental.pallas import tpu as pltpu

_W = 200
_H = 200
_A = 9
_NH = 5
_G = 20
_N = _NH * _A * _W * _H
_PER_HEAD = _A * _W * _H
_BIG = 1 << 30
_C7 = float(np.float32(0.7 / 1.7))
_C3 = float(np.float32(0.3 / 1.3))


_WC = 40


def _main_body(gt_ref, cls_ref, reg_ref, ints_ref, floats_ref, wacc_ref):
    i = pl.program_id(0)
    head = i // _A
    a = i - head * _A

    gts = [[gt_ref[g, c] for c in range(4)] for g in range(_G)]
    area_g = [(gts[g][2] - gts[g][0]) * (gts[g][3] - gts[g][1]) for g in range(_G)]

    w2d = lax.broadcasted_iota(jnp.int32, (_WC, _H), 0)
    h2d = lax.broadcasted_iota(jnp.int32, (_WC, _H), 1)
    base_off = head * _PER_HEAD + a

    sum_b = None
    npos_s = None
    nzero_s = None
    fb_s = None
    for ws in range(0, _W, _WC):
        sl = slice(ws, ws + _WC)
        p = cls_ref[0, 0, 0, sl, :]
        bx1 = reg_ref[0, 0, 0, sl, :]
        by1 = reg_ref[0, 0, 1, sl, :]
        bx2 = reg_ref[0, 0, 2, sl, :]
        by2 = reg_ref[0, 0, 3, sl, :]
        area_b = (bx2 - bx1) * (by2 - by1)
        n_c = ((w2d + ws) * _H + h2d) * _A + base_off

        jf = None
        nviol = None
        for g in range(_G):
            gx1, gy1, gx2, gy2 = gts[g]
            iw = jnp.maximum(jnp.minimum(bx2, gx2) - jnp.maximum(bx1, gx1), 0.0)
            ih = jnp.maximum(jnp.minimum(by2, gy2) - jnp.maximum(by1, gy1), 0.0)
            inter = iw * ih
            ab = area_b + area_g[g]
            upos = inter < ab
            pos = (inter >= _C7 * ab) & upos
            viol = (inter >= _C3 * ab) & upos
            inc = jnp.where(viol, 1, 0)
            nviol = inc if nviol is None else (nviol + inc)
            reach = (jf >= g) if jf is not None else None
            cand = (inter > 0.0) & upos
            if reach is not None:
                cand = cand & reach
            sel = jnp.where(cand, n_c, -1)
            if ws == 0:
                wacc_ref[g] = sel
            else:
                wacc_ref[g] = jnp.maximum(wacc_ref[g], sel)
            posg = jnp.where(pos, g, _G)
            jf = posg if jf is None else jnp.minimum(jf, posg)

        anypos = jf < _G
        zerolab = (jf >= _G) & (nviol == 0)
        basekeep = anypos | zerolab
        bce = jnp.where(anypos, -jnp.log(p), 0.0) + jnp.where(
            zerolab, -jnp.log(1.0 - p), 0.0)
        t = jnp.sum(bce)
        sum_b = t if sum_b is None else (sum_b + t)
        t = jnp.sum(jnp.where(anypos, 1, 0))
        npos_s = t if npos_s is None else (npos_s + t)
        t = jnp.sum(jnp.where(zerolab, 1, 0))
        nzero_s = t if nzero_s is None else (nzero_s + t)
        t = jnp.min(jnp.where(basekeep, n_c, _BIG))
        fb_s = t if fb_s is None else jnp.minimum(fb_s, t)

    for g in range(_G):
        ints_ref[0, 0, g] = jnp.max(wacc_ref[g])
    ints_ref[0, 0, _G] = fb_s
    ints_ref[0, 0, _G + 1] = npos_s
    ints_ref[0, 0, _G + 2] = nzero_s
    floats_ref[0, 0, 0] = sum_b


def _combine_body(ints_ref, floats_ref, pref_ref, fvals_ref):
    nblk = _NH * _A
    iv = ints_ref[...].reshape(nblk, 32)
    fv = floats_ref[...].reshape(nblk, 8)
    maxv = jnp.max(iv, axis=0, keepdims=True)
    minv = jnp.min(iv, axis=0, keepdims=True)
    sumv = jnp.sum(iv, axis=0, keepdims=True)
    sumf = jnp.sum(fv, axis=0, keepdims=True)
    for g in range(_G):
        pref_ref[g] = jnp.maximum(maxv[0, g], 0)
    fb = minv[0, _G]
    pref_ref[_G] = jnp.minimum(fb, _N - 1)
    pref_ref[_G + 1] = fb
    pref_ref[_G + 2] = sumv[0, _G + 1] + sumv[0, _G + 2]
    fvals_ref[0, 0] = sumf[0, 0]


def _epi_body(pref_ref, gt_ref, fvals_ref, cls_ref, reg_ref, out_ref, scr_f):
    k_step = pl.program_id(0)

    gts = [[gt_ref[g, c] for c in range(4)] for g in range(_G)]
    area_g = [(gts[g][2] - gts[g][0]) * (gts[g][3] - gts[g][1]) for g in range(_G)]

    w2d = lax.broadcasted_iota(jnp.int32, (_W, _H), 0)
    h2d = lax.broadcasted_iota(jnp.int32, (_W, _H), 1)
    whbase = w2d * _H + h2d
    idx = pref_ref[k_step]
    rem = idx - (idx // _PER_HEAD) * _PER_HEAD
    wh = rem // _A
    eq = whbase == wh
    for c in range(4):
        scr_f[8 + 4 * k_step + c] = jnp.sum(jnp.where(eq, reg_ref[0, 0, c], 0.0))

    @pl.when(k_step < _G)
    def _extract_p():
        p = jnp.sum(jnp.where(eq, cls_ref[0, 0, 0], 0.0))
        pv = jnp.full((1, 128), p, dtype=jnp.float32)
        scr_f[100 + k_step] = -jnp.log(pv)[0, 0]
        scr_f[128 + k_step] = -jnp.log(1.0 - pv)[0, 0]

    @pl.when(k_step == _G)
    def _assemble():
        w_use = [pref_ref[k] for k in range(_G)]
        fb = pref_ref[_G + 1]
        sum_b = fvals_ref[0, 0]
        nk_extra = None
        for g in range(_G):
            dup = None
            for gp in range(g):
                t = w_use[g] == w_use[gp]
                dup = t if dup is None else (dup | t)
            bx1 = scr_f[8 + 4 * g + 0]
            by1 = scr_f[8 + 4 * g + 1]
            bx2 = scr_f[8 + 4 * g + 2]
            by2 = scr_f[8 + 4 * g + 3]
            area_b = (bx2 - bx1) * (by2 - by1)
            anypos = None
            alllow = None
            for j in range(_G):
                gx1, gy1, gx2, gy2 = gts[j]
                iw = jnp.maximum(jnp.minimum(bx2, gx2) - jnp.maximum(bx1, gx1), 0.0)
                ih = jnp.maximum(jnp.minimum(by2, gy2) - jnp.maximum(by1, gy1), 0.0)
                inter = iw * ih
                ab = area_b + area_g[j]
                nupos = inter >= ab
                pos = (inter >= _C7 * ab) & (~nupos)
                low = (inter < _C3 * ab) | nupos
                anypos = pos if anypos is None else (anypos | pos)
                alllow = low if alllow is None else (alllow & low)
            zerolab = (~anypos) & alllow
            add1 = (~anypos) & (~zerolab)
            repl = zerolab
            if dup is not None:
                add1 = (~dup) & add1
                repl = (~dup) & repl
            lp = scr_f[100 + g]
            l1m = scr_f[128 + g]
            sum_b = sum_b + jnp.where(add1, lp, 0.0) + jnp.where(
                repl, lp - l1m, 0.0)
            inc = jnp.where(add1, 1, 0)
            nk_extra = inc if nk_extra is None else (nk_extra + inc)

        n_keep = (pref_ref[_G + 2] + nk_extra).astype(jnp.float32)
        num = jnp.full((1, 128), sum_b, dtype=jnp.float32)
        den = jnp.full((1, 128), n_keep, dtype=jnp.float32)
        cls_loss = (num / den)[0, 0]

        first_idx = fb
        for g in range(_G):
            first_idx = jnp.minimum(first_idx, w_use[g])
        box0 = [None] * 4
        found = None
        cands = [(w_use[g], g) for g in range(_G)] + [(fb, _G)]
        for idx_k, k in cands:
            sel = idx_k == first_idx
            if found is not None:
                sel = sel & (~found)
            for c in range(4):
                v = scr_f[8 + 4 * k + c]
                box0[c] = jnp.where(sel, v, 0.0) if box0[c] is None else \
                    jnp.where(sel, v, box0[c])
            found = sel if found is None else (found | sel)

        acc = None
        for j in range(_G):
            for c in range(4):
                d = jnp.abs(box0[c] - gts[j][c])
                t = jnp.where(d < 1.0, 0.5 * d * d, d - 0.5)
                acc = t if acc is None else (acc + t)
        reg_loss = acc * (1.0 / (4 * _G))
        out_ref[0, 0] = cls_loss * (1.0 / 256.0) + reg_loss * (1.0 / 2400.0)


def kernel(cls_heads, reg_heads, gt_boxes, interpret=False):
    nblk = _NH * _A
    ints, floats = pl.pallas_call(
        _main_body,
        grid=(nblk,),
        in_specs=[
            pl.BlockSpec(memory_space=pltpu.SMEM),
            pl.BlockSpec((1, 1, 1, _W, _H), lambda i: (i // _A, 0, i % _A, 0, 0)),
            pl.BlockSpec((1, 1, 4, _W, _H), lambda i: (i // _A, 0, i % _A, 0, 0)),
        ],
        out_specs=[
            pl.BlockSpec((1, 1, 32), lambda i: (i, 0, 0), memory_space=pltpu.SMEM),
            pl.BlockSpec((1, 1, 8), lambda i: (i, 0, 0), memory_space=pltpu.SMEM),
        ],
        out_shape=[
            jax.ShapeDtypeStruct((nblk, 1, 32), jnp.int32),
            jax.ShapeDtypeStruct((nblk, 1, 8), jnp.float32),
        ],
        scratch_shapes=[pltpu.VMEM((_G, _WC, _H), jnp.int32)],
        compiler_params=pltpu.CompilerParams(
            dimension_semantics=("arbitrary",),
        ),
        name="rpn_main",
        interpret=interpret,
    )(gt_boxes, cls_heads, reg_heads)

    pref, fvals = pl.pallas_call(
        _combine_body,
        in_specs=[
            pl.BlockSpec(memory_space=pltpu.VMEM),
            pl.BlockSpec(memory_space=pltpu.VMEM),
        ],
        out_specs=[
            pl.BlockSpec(memory_space=pltpu.SMEM),
            pl.BlockSpec(memory_space=pltpu.SMEM),
        ],
        out_shape=[
            jax.ShapeDtypeStruct((32,), jnp.int32),
            jax.ShapeDtypeStruct((1, 8), jnp.float32),
        ],
        name="rpn_combine",
        interpret=interpret,
    )(ints, floats)

    def _cls_map(k, pref):
        idx = pref[k]
        return (idx // _PER_HEAD, 0, (idx - (idx // _PER_HEAD) * _PER_HEAD) % _A,
                0, 0)

    out = pl.pallas_call(
        _epi_body,
        grid_spec=pltpu.PrefetchScalarGridSpec(
            num_scalar_prefetch=1,
            grid=(_G + 1,),
            in_specs=[
                pl.BlockSpec(memory_space=pltpu.SMEM),
                pl.BlockSpec(memory_space=pltpu.SMEM),
                pl.BlockSpec((1, 1, 1, _W, _H), _cls_map),
                pl.BlockSpec((1, 1, 4, _W, _H), _cls_map),
            ],
            out_specs=pl.BlockSpec((1, 1), lambda k, pref: (0, 0),
                                   memory_space=pltpu.SMEM),
            scratch_shapes=[pltpu.SMEM((256,), jnp.float32)],
        ),
        out_shape=jax.ShapeDtypeStruct((1, 1), jnp.float32),
        compiler_params=pltpu.CompilerParams(
            dimension_semantics=("arbitrary",),
        ),
        name="rpn_epilogue",
        interpret=interpret,
    )(pref, gt_boxes, fvals, cls_heads, reg_heads)
    return out[0, 0]


# speedup vs baseline: 1.1471x; 1.0410x over previous
import jax
import jax.numpy as jnp
import numpy as np
from jax import lax
from jax.experimental import pallas as pl
from jax.experimental.pallas import tpu as pltpu

_W = 200
_H = 200
_A = 9
_NH = 5
_G = 20
_N = _NH * _A * _W * _H
_PER_HEAD = _A * _W * _H
_BIG = 1 << 30
_C7 = float(np.float32(0.7 / 1.7))
_C3 = float(np.float32(0.3 / 1.3))


_WC = 40


def _main_body(gt_ref, cls_ref, reg_ref, ints_ref, floats_ref):
    i = pl.program_id(0)
    head = i // _A
    a = i - head * _A

    gts = [[gt_ref[g, c] for c in range(4)] for g in range(_G)]
    area_g = [(gts[g][2] - gts[g][0]) * (gts[g][3] - gts[g][1]) for g in range(_G)]

    w2d = lax.broadcasted_iota(jnp.int32, (_WC, _H), 0)
    h2d = lax.broadcasted_iota(jnp.int32, (_WC, _H), 1)
    base_off = head * _PER_HEAD + a

    winners = [None] * _G
    sum_b = None
    npos_s = None
    nzero_s = None
    fb_s = None
    for ws in range(0, _W, _WC):
        sl = slice(ws, ws + _WC)
        p = cls_ref[0, 0, 0, sl, :]
        bx1 = reg_ref[0, 0, 0, sl, :]
        by1 = reg_ref[0, 0, 1, sl, :]
        bx2 = reg_ref[0, 0, 2, sl, :]
        by2 = reg_ref[0, 0, 3, sl, :]
        area_b = (bx2 - bx1) * (by2 - by1)
        n_c = ((w2d + ws) * _H + h2d) * _A + base_off

        jf = None
        nviol = None
        for g in range(_G):
            gx1, gy1, gx2, gy2 = gts[g]
            iw = jnp.maximum(jnp.minimum(bx2, gx2) - jnp.maximum(bx1, gx1), 0.0)
            ih = jnp.maximum(jnp.minimum(by2, gy2) - jnp.maximum(by1, gy1), 0.0)
            inter = iw * ih
            ab = area_b + area_g[g]
            upos = inter < ab
            pos = (inter >= _C7 * ab) & upos
            viol = (inter >= _C3 * ab) & upos
            inc = jnp.where(viol, 1, 0)
            nviol = inc if nviol is None else (nviol + inc)
            reach = (jf >= g) if jf is not None else None
            cand = (inter > 0.0) & upos
            if reach is not None:
                cand = cand & reach
            m_cg = jnp.max(jnp.where(cand, n_c, -1))
            winners[g] = m_cg if winners[g] is None else \
                jnp.maximum(winners[g], m_cg)
            posg = jnp.where(pos, g, _G)
            jf = posg if jf is None else jnp.minimum(jf, posg)

        anypos = jf < _G
        zerolab = (jf >= _G) & (nviol == 0)
        basekeep = anypos | zerolab
        bce = jnp.where(anypos, -jnp.log(p), 0.0) + jnp.where(
            zerolab, -jnp.log(1.0 - p), 0.0)
        t = jnp.sum(bce)
        sum_b = t if sum_b is None else (sum_b + t)
        t = jnp.sum(jnp.where(anypos, 1, 0))
        npos_s = t if npos_s is None else (npos_s + t)
        t = jnp.sum(jnp.where(zerolab, 1, 0))
        nzero_s = t if nzero_s is None else (nzero_s + t)
        t = jnp.min(jnp.where(basekeep, n_c, _BIG))
        fb_s = t if fb_s is None else jnp.minimum(fb_s, t)

    for g in range(_G):
        ints_ref[0, 0, g] = winners[g]
    ints_ref[0, 0, _G] = fb_s
    ints_ref[0, 0, _G + 1] = npos_s
    ints_ref[0, 0, _G + 2] = nzero_s
    floats_ref[0, 0, 0] = sum_b


def _combine_body(ints_ref, floats_ref, pref_ref, fvals_ref):
    nblk = _NH * _A
    iv = ints_ref[...].reshape(nblk, 32)
    fv = floats_ref[...].reshape(nblk, 8)
    maxv = jnp.max(iv, axis=0, keepdims=True)
    minv = jnp.min(iv, axis=0, keepdims=True)
    sumv = jnp.sum(iv, axis=0, keepdims=True)
    sumf = jnp.sum(fv, axis=0, keepdims=True)
    for g in range(_G):
        pref_ref[g] = jnp.maximum(maxv[0, g], 0)
    fb = minv[0, _G]
    pref_ref[_G] = jnp.minimum(fb, _N - 1)
    pref_ref[_G + 1] = fb
    pref_ref[_G + 2] = sumv[0, _G + 1] + sumv[0, _G + 2]
    fvals_ref[0, 0] = sumf[0, 0]


def _epi_body(pref_ref, gt_ref, fvals_ref, cls_ref, reg_ref, out_ref, scr_f):
    k_step = pl.program_id(0)

    gts = [[gt_ref[g, c] for c in range(4)] for g in range(_G)]
    area_g = [(gts[g][2] - gts[g][0]) * (gts[g][3] - gts[g][1]) for g in range(_G)]

    w2d = lax.broadcasted_iota(jnp.int32, (_W, _H), 0)
    h2d = lax.broadcasted_iota(jnp.int32, (_W, _H), 1)
    whbase = w2d * _H + h2d
    idx = pref_ref[k_step]
    rem = idx - (idx // _PER_HEAD) * _PER_HEAD
    wh = rem // _A
    eq = whbase == wh
    for c in range(4):
        scr_f[8 + 4 * k_step + c] = jnp.sum(jnp.where(eq, reg_ref[0, 0, c], 0.0))

    @pl.when(k_step < _G)
    def _extract_p():
        p = jnp.sum(jnp.where(eq, cls_ref[0, 0, 0], 0.0))
        pv = jnp.full((1, 128), p, dtype=jnp.float32)
        scr_f[100 + k_step] = -jnp.log(pv)[0, 0]
        scr_f[128 + k_step] = -jnp.log(1.0 - pv)[0, 0]

    @pl.when(k_step == _G)
    def _assemble():
        w_use = [pref_ref[k] for k in range(_G)]
        fb = pref_ref[_G + 1]
        sum_b = fvals_ref[0, 0]
        nk_extra = None
        for g in range(_G):
            dup = None
            for gp in range(g):
                t = w_use[g] == w_use[gp]
                dup = t if dup is None else (dup | t)
            bx1 = scr_f[8 + 4 * g + 0]
            by1 = scr_f[8 + 4 * g + 1]
            bx2 = scr_f[8 + 4 * g + 2]
            by2 = scr_f[8 + 4 * g + 3]
            area_b = (bx2 - bx1) * (by2 - by1)
            anypos = None
            alllow = None
            for j in range(_G):
                gx1, gy1, gx2, gy2 = gts[j]
                iw = jnp.maximum(jnp.minimum(bx2, gx2) - jnp.maximum(bx1, gx1), 0.0)
                ih = jnp.maximum(jnp.minimum(by2, gy2) - jnp.maximum(by1, gy1), 0.0)
                inter = iw * ih
                ab = area_b + area_g[j]
                nupos = inter >= ab
                pos = (inter >= _C7 * ab) & (~nupos)
                low = (inter < _C3 * ab) | nupos
                anypos = pos if anypos is None else (anypos | pos)
                alllow = low if alllow is None else (alllow & low)
            zerolab = (~anypos) & alllow
            add1 = (~anypos) & (~zerolab)
            repl = zerolab
            if dup is not None:
                add1 = (~dup) & add1
                repl = (~dup) & repl
            lp = scr_f[100 + g]
            l1m = scr_f[128 + g]
            sum_b = sum_b + jnp.where(add1, lp, 0.0) + jnp.where(
                repl, lp - l1m, 0.0)
            inc = jnp.where(add1, 1, 0)
            nk_extra = inc if nk_extra is None else (nk_extra + inc)

        n_keep = (pref_ref[_G + 2] + nk_extra).astype(jnp.float32)
        num = jnp.full((1, 128), sum_b, dtype=jnp.float32)
        den = jnp.full((1, 128), n_keep, dtype=jnp.float32)
        cls_loss = (num / den)[0, 0]

        first_idx = fb
        for g in range(_G):
            first_idx = jnp.minimum(first_idx, w_use[g])
        box0 = [None] * 4
        found = None
        cands = [(w_use[g], g) for g in range(_G)] + [(fb, _G)]
        for idx_k, k in cands:
            sel = idx_k == first_idx
            if found is not None:
                sel = sel & (~found)
            for c in range(4):
                v = scr_f[8 + 4 * k + c]
                box0[c] = jnp.where(sel, v, 0.0) if box0[c] is None else \
                    jnp.where(sel, v, box0[c])
            found = sel if found is None else (found | sel)

        acc = None
        for j in range(_G):
            for c in range(4):
                d = jnp.abs(box0[c] - gts[j][c])
                t = jnp.where(d < 1.0, 0.5 * d * d, d - 0.5)
                acc = t if acc is None else (acc + t)
        reg_loss = acc * (1.0 / (4 * _G))
        out_ref[0, 0] = cls_loss * (1.0 / 256.0) + reg_loss * (1.0 / 2400.0)


def kernel(cls_heads, reg_heads, gt_boxes, interpret=False):
    nblk = _NH * _A
    ints, floats = pl.pallas_call(
        _main_body,
        grid=(nblk,),
        in_specs=[
            pl.BlockSpec(memory_space=pltpu.SMEM),
            pl.BlockSpec((1, 1, 1, _W, _H), lambda i: (i // _A, 0, i % _A, 0, 0)),
            pl.BlockSpec((1, 1, 4, _W, _H), lambda i: (i // _A, 0, i % _A, 0, 0)),
        ],
        out_specs=[
            pl.BlockSpec((1, 1, 32), lambda i: (i, 0, 0), memory_space=pltpu.SMEM),
            pl.BlockSpec((1, 1, 8), lambda i: (i, 0, 0), memory_space=pltpu.SMEM),
        ],
        out_shape=[
            jax.ShapeDtypeStruct((nblk, 1, 32), jnp.int32),
            jax.ShapeDtypeStruct((nblk, 1, 8), jnp.float32),
        ],
        compiler_params=pltpu.CompilerParams(
            dimension_semantics=("arbitrary",),
        ),
        name="rpn_main",
        interpret=interpret,
    )(gt_boxes, cls_heads, reg_heads)

    pref, fvals = pl.pallas_call(
        _combine_body,
        in_specs=[
            pl.BlockSpec(memory_space=pltpu.VMEM),
            pl.BlockSpec(memory_space=pltpu.VMEM),
        ],
        out_specs=[
            pl.BlockSpec(memory_space=pltpu.SMEM),
            pl.BlockSpec(memory_space=pltpu.SMEM),
        ],
        out_shape=[
            jax.ShapeDtypeStruct((32,), jnp.int32),
            jax.ShapeDtypeStruct((1, 8), jnp.float32),
        ],
        name="rpn_combine",
        interpret=interpret,
    )(ints, floats)

    def _cls_map(k, pref):
        idx = pref[k]
        return (idx // _PER_HEAD, 0, (idx - (idx // _PER_HEAD) * _PER_HEAD) % _A,
                0, 0)

    out = pl.pallas_call(
        _epi_body,
        grid_spec=pltpu.PrefetchScalarGridSpec(
            num_scalar_prefetch=1,
            grid=(_G + 1,),
            in_specs=[
                pl.BlockSpec(memory_space=pltpu.SMEM),
                pl.BlockSpec(memory_space=pltpu.SMEM),
                pl.BlockSpec((1, 1, 1, _W, _H), _cls_map),
                pl.BlockSpec((1, 1, 4, _W, _H), _cls_map),
            ],
            out_specs=pl.BlockSpec((1, 1), lambda k, pref: (0, 0),
                                   memory_space=pltpu.SMEM),
            scratch_shapes=[pltpu.SMEM((256,), jnp.float32)],
        ),
        out_shape=jax.ShapeDtypeStruct((1, 1), jnp.float32),
        compiler_params=pltpu.CompilerParams(
            dimension_semantics=("arbitrary",),
        ),
        name="rpn_epilogue",
        interpret=interpret,
    )(pref, gt_boxes, fvals, cls_heads, reg_heads)
    return out[0, 0]
